# baseline (speedup 1.0000x reference)
import functools

import jax
import jax.numpy as jnp
from jax import lax
from jax.experimental import pallas as pl
from jax.experimental.pallas import tpu as pltpu
from jax.experimental.pallas import tpu_sc as plsc

N = 10000
E = 320000
D = 128
C = 40
D2 = 48
NC, NS, L = 2, 16, 16
NW = NC * NS
B = 128
EPW = 10240
NBLK = EPW // B
NPAD = 10240
SLAB = NPAD // NS
RBLK = 2000


def _zero_rows(ref, nrows, width):
    zv = jnp.zeros((L,), jnp.float32)

    def body(i, _):
        for k in range(width // L):
            ref[i, pl.ds(k * L, L)] = zv
        return 0

    lax.fori_loop(0, nrows, body, 0)


def _sc_mesh():
    return plsc.VectorSubcoreMesh(core_axis_name="c", subcore_axis_name="s")


def _sc_degrees(dst_p, ew_p, one_p):
    @functools.partial(
        pl.kernel,
        out_type=jax.ShapeDtypeStruct((NC, 2, NPAD), jnp.float32),
        mesh=_sc_mesh(),
        scratch_types=[
            pltpu.VMEM((NBLK, B), jnp.int32),
            pltpu.VMEM((NBLK, B), jnp.float32),
            pltpu.VMEM((NBLK, B), jnp.float32),
            pltpu.VMEM((SLAB,), jnp.float32),
            pltpu.VMEM_SHARED((NPAD,), jnp.float32),
            pltpu.VMEM_SHARED((NPAD,), jnp.float32),
        ],
    )
    def deg_k(dst_h, ew_h, one_h, out_h, dst_v, ew_v, one_v, zbuf, acc1_sh, acc2_sh):
        lc = lax.axis_index("c")
        ls = lax.axis_index("s")
        wid = ls * NC + lc

        zv = jnp.zeros((L,), jnp.float32)

        def zb(i, _):
            zbuf[pl.ds(i * L, L)] = zv
            return 0

        lax.fori_loop(0, SLAB // L, zb, 0)
        sl = pl.ds(ls * SLAB, SLAB)
        pltpu.sync_copy(zbuf, acc1_sh.at[sl])
        pltpu.sync_copy(zbuf, acc2_sh.at[sl])

        pltpu.sync_copy(dst_h.at[wid], dst_v)
        pltpu.sync_copy(ew_h.at[wid], ew_v)
        pltpu.sync_copy(one_h.at[wid], one_v)
        plsc.subcore_barrier()

        def blk(j, _):
            idx = dst_v.at[j]
            pltpu.sync_copy(ew_v.at[j], acc1_sh.at[idx], add=True)
            pltpu.sync_copy(one_v.at[j], acc2_sh.at[idx], add=True)
            return 0

        lax.fori_loop(0, NBLK, blk, 0)
        plsc.subcore_barrier()

        pltpu.sync_copy(acc1_sh.at[sl], out_h.at[lc, 0, sl])
        pltpu.sync_copy(acc2_sh.at[sl], out_h.at[lc, 1, sl])

    return deg_k(dst_p, ew_p, one_p)


NB = 4


def _sc_aggregate(src_p, dst_p, ew_p, feat, width):
    scratch = [
        pltpu.VMEM((NBLK, B), jnp.int32),
        pltpu.VMEM((NBLK, B), jnp.float32),
        pltpu.VMEM((NBLK, B), jnp.int32),
        pltpu.VMEM((NB, B, width), jnp.float32),
        pltpu.VMEM_SHARED((NPAD, width), jnp.float32),
    ] + [pltpu.SemaphoreType.DMA] * (2 * NB)
    scale = ew_p is not None
    params = (
        None
        if width % 128 == 0
        else pltpu.CompilerParams(use_tc_tiling_on_sc=False)
    )

    @functools.partial(
        pl.kernel,
        out_type=jax.ShapeDtypeStruct((NC, NPAD, width), jnp.float32),
        mesh=_sc_mesh(),
        scratch_types=scratch,
        compiler_params=params,
    )
    def agg_k(src_h, ew_h, dst_h, feat_h, out_h, src_v, ew_v, dst_v, rows, acc_sh, *sems):
        gsem = sems[:NB]
        ssem = sems[NB:]
        lc = lax.axis_index("c")
        ls = lax.axis_index("s")
        wid = ls * NC + lc

        zv = jnp.zeros((L,), jnp.float32)

        def zb(i, _):
            for k in range(width // L):
                rows[0, i, pl.ds(k * L, L)] = zv
            return 0

        lax.fori_loop(0, B, zb, 0)

        def zs(t, _):
            pltpu.sync_copy(rows.at[0], acc_sh.at[pl.ds(ls * SLAB + t * B, B)])
            return 0

        lax.fori_loop(0, SLAB // B, zs, 0)

        pltpu.sync_copy(src_h.at[wid], src_v)
        pltpu.sync_copy(dst_h.at[wid], dst_v)
        if scale:
            pltpu.sync_copy(ew_h.at[wid], ew_v)
        plsc.subcore_barrier()

        def g_issue(j, b):
            pltpu.async_copy(feat_h.at[src_v.at[j]], rows.at[b], gsem[b])

        def g_wait(j, b):
            pltpu.make_async_copy(feat_h.at[src_v.at[j]], rows.at[b], gsem[b]).wait()

        def s_issue(j, b):
            pltpu.async_copy(rows.at[b], acc_sh.at[dst_v.at[j]], ssem[b], add=True)

        def s_wait(j, b):
            pltpu.make_async_copy(rows.at[b], acc_sh.at[dst_v.at[j]], ssem[b]).wait()

        def do_scale(j, b):
            def scale_g(g, _):
                base = g * L
                wvec = ew_v[j, pl.ds(base, L)]
                for e in range(L):
                    w = wvec[e]
                    for k in range(width // L):
                        sl = pl.ds(k * L, L)
                        rows[b, base + e, sl] = rows[b, base + e, sl] * w
                return 0

            lax.fori_loop(0, B // L, scale_g, 0)

        g_issue(0, 0)

        def turn(t, _):
            for b in range(NB):
                j = t * NB + b
                bn = (b + 1) % NB

                @pl.when(j >= NB - 1)
                def _():
                    s_wait(j - (NB - 1), bn)

                @pl.when(j <= NBLK - 2)
                def _():
                    g_issue(j + 1, bn)

                g_wait(j, b)
                if scale:
                    do_scale(j, b)
                s_issue(j, b)
            return 0

        lax.fori_loop(0, NBLK // NB, turn, 0)
        for j in range(NBLK - NB + 1, NBLK):
            s_wait(j, j % NB)
        plsc.subcore_barrier()

        def co(t, _):
            sl = pl.ds(ls * SLAB + t * B, B)
            pltpu.sync_copy(acc_sh.at[sl], out_h.at[lc, sl])
            return 0

        lax.fori_loop(0, SLAB // B, co, 0)

    if ew_p is None:
        ew_p = src_p
    return agg_k(src_p, ew_p, dst_p, feat)


def _tc1(x, W1, degp):
    def body(x_ref, w_ref, degp_ref, hpa_ref, hpb_ref, d1_ref, d2_ref):
        dgp = degp_ref[...]
        deg1 = dgp[0] + dgp[2] + 1.0
        deg2 = dgp[1] + dgp[3] + 1.0
        d1 = lax.rsqrt(deg1)
        d2 = lax.rsqrt(deg2)
        h = jnp.dot(x_ref[...], w_ref[...], preferred_element_type=jnp.float32)
        hp = h * d1
        hpa_ref[...] = hp[:, : D // 2]
        hpb_ref[...] = hp[:, D // 2 :]
        d1_ref[...] = d1
        d2_ref[...] = d2

    return pl.pallas_call(
        body,
        grid=(N // RBLK,),
        in_specs=[
            pl.BlockSpec((RBLK, D), lambda i: (i, 0)),
            pl.BlockSpec((D, D), lambda i: (0, 0)),
            pl.BlockSpec((NC * 2, RBLK, 1), lambda i: (0, i, 0)),
        ],
        out_specs=[
            pl.BlockSpec((RBLK, D // 2), lambda i: (i, 0)),
            pl.BlockSpec((RBLK, D // 2), lambda i: (i, 0)),
            pl.BlockSpec((RBLK, 1), lambda i: (i, 0)),
            pl.BlockSpec((RBLK, 1), lambda i: (i, 0)),
        ],
        out_shape=[
            jax.ShapeDtypeStruct((N, D // 2), jnp.float32),
            jax.ShapeDtypeStruct((N, D // 2), jnp.float32),
            jax.ShapeDtypeStruct((N, 1), jnp.float32),
            jax.ShapeDtypeStruct((N, 1), jnp.float32),
        ],
    )(x, W1, degp)


def _tc2(acc1a, acc1b, hpa, hpb, d1, d2, b1, W2p):
    def body(aa_ref, ab_ref, hpa_ref, hpb_ref, d1_ref, d2_ref, b1_ref, w2_ref, h2p_ref):
        aa = aa_ref[...]
        ab = ab_ref[...]
        acc = jnp.concatenate([aa[0] + aa[1], ab[0] + ab[1]], axis=1)
        hp = jnp.concatenate([hpa_ref[...], hpb_ref[...]], axis=1)
        out1 = jnp.maximum(d1_ref[...] * (acc + hp) + b1_ref[...], 0.0)
        h2 = jnp.dot(out1, w2_ref[...], preferred_element_type=jnp.float32)
        h2p_ref[...] = h2 * d2_ref[...]

    half = pl.BlockSpec((NC, RBLK, D // 2), lambda i: (0, i, 0))
    feat = pl.BlockSpec((RBLK, D // 2), lambda i: (i, 0))
    return pl.pallas_call(
        body,
        grid=(N // RBLK,),
        in_specs=[
            half,
            half,
            feat,
            feat,
            pl.BlockSpec((RBLK, 1), lambda i: (i, 0)),
            pl.BlockSpec((RBLK, 1), lambda i: (i, 0)),
            pl.BlockSpec((1, D), lambda i: (0, 0)),
            pl.BlockSpec((D, D2), lambda i: (0, 0)),
        ],
        out_specs=pl.BlockSpec((RBLK, D2), lambda i: (i, 0)),
        out_shape=jax.ShapeDtypeStruct((N, D2), jnp.float32),
    )(acc1a, acc1b, hpa, hpb, d1, d2, b1, W2p)


def _tc3(acc2, h2p, d2, b2p):
    def body(acc_ref, h2p_ref, d2_ref, b2_ref, out_ref):
        a = acc_ref[...]
        full = d2_ref[...] * (a[0] + a[1] + h2p_ref[...]) + b2_ref[...]
        out_ref[...] = full[:, :C]

    return pl.pallas_call(
        body,
        grid=(N // RBLK,),
        in_specs=[
            pl.BlockSpec((NC, RBLK, D2), lambda i: (0, i, 0)),
            pl.BlockSpec((RBLK, D2), lambda i: (i, 0)),
            pl.BlockSpec((RBLK, 1), lambda i: (i, 0)),
            pl.BlockSpec((1, D2), lambda i: (0, 0)),
        ],
        out_specs=pl.BlockSpec((RBLK, C), lambda i: (i, 0)),
        out_shape=jax.ShapeDtypeStruct((N, C), jnp.float32),
    )(acc2, h2p, d2, b2p)


def kernel(x, edge_index, edge_weight, W1, b1, W2, b2):
    src = edge_index[0].astype(jnp.int32)
    dst = edge_index[1].astype(jnp.int32)
    ew = edge_weight.astype(jnp.float32)

    ppw = EPW - E // NW
    pad_dst = jnp.broadcast_to(N + jnp.arange(ppw, dtype=jnp.int32), (NW, ppw))

    def _pack_i(a, pad_val):
        return jnp.concatenate(
            [a.reshape(NW, E // NW), pad_val], axis=1
        ).reshape(NW, NBLK, B)

    src_p = _pack_i(src, jnp.zeros((NW, ppw), jnp.int32))
    dst_p = _pack_i(dst, pad_dst)
    ew_p = _pack_i(ew, jnp.zeros((NW, ppw), jnp.float32))
    one_p = _pack_i(jnp.ones((E,), jnp.float32), jnp.zeros((NW, ppw), jnp.float32))

    W2p = jnp.pad(W2.astype(jnp.float32), ((0, 0), (0, D2 - C)))
    b1r = b1.astype(jnp.float32).reshape(1, D)
    b2p = jnp.pad(b2.astype(jnp.float32), (0, D2 - C)).reshape(1, D2)

    degp = _sc_degrees(dst_p, ew_p, one_p)
    degp4 = degp.reshape(NC * 2, NPAD)[:, :, None]
    hpa, hpb, d1, d2 = _tc1(x.astype(jnp.float32), W1.astype(jnp.float32), degp4)
    acc1a = _sc_aggregate(src_p, dst_p, ew_p, hpa, D // 2)
    acc1b = _sc_aggregate(src_p, dst_p, ew_p, hpb, D // 2)
    h2p = _tc2(acc1a, acc1b, hpa, hpb, d1, d2, b1r, W2p)
    acc2 = _sc_aggregate(src_p, dst_p, None, h2p, D2)
    return _tc3(acc2, h2p, d2, b2p)


# speedup vs baseline: 1.3278x; 1.3278x over previous
import functools

import jax
import jax.numpy as jnp
from jax import lax
from jax.experimental import pallas as pl
from jax.experimental.pallas import tpu as pltpu
from jax.experimental.pallas import tpu_sc as plsc

N = 10000
E = 320000
D = 128
C = 40
D2 = 48
NC, NS, L = 2, 16, 16
NW = NC * NS
B = 128
EPW = 10240
NBLK = EPW // B
NPAD = 10240
SLAB = NPAD // NS
RBLK = 2000


def _zero_rows(ref, nrows, width):
    zv = jnp.zeros((L,), jnp.float32)

    def body(i, _):
        for k in range(width // L):
            ref[i, pl.ds(k * L, L)] = zv
        return 0

    lax.fori_loop(0, nrows, body, 0)


def _sc_mesh():
    return plsc.VectorSubcoreMesh(core_axis_name="c", subcore_axis_name="s")


def _sc_degrees(dst_p, ew_p, one_p):
    @functools.partial(
        pl.kernel,
        out_type=jax.ShapeDtypeStruct((NC, 2, NPAD), jnp.float32),
        mesh=_sc_mesh(),
        scratch_types=[
            pltpu.VMEM((NBLK, B), jnp.int32),
            pltpu.VMEM((NBLK, B), jnp.float32),
            pltpu.VMEM((NBLK, B), jnp.float32),
            pltpu.VMEM((SLAB,), jnp.float32),
            pltpu.VMEM_SHARED((NPAD,), jnp.float32),
            pltpu.VMEM_SHARED((NPAD,), jnp.float32),
        ],
    )
    def deg_k(dst_h, ew_h, one_h, out_h, dst_v, ew_v, one_v, zbuf, acc1_sh, acc2_sh):
        lc = lax.axis_index("c")
        ls = lax.axis_index("s")
        wid = ls * NC + lc

        zv = jnp.zeros((L,), jnp.float32)

        def zb(i, _):
            zbuf[pl.ds(i * L, L)] = zv
            return 0

        lax.fori_loop(0, SLAB // L, zb, 0)
        sl = pl.ds(ls * SLAB, SLAB)
        pltpu.sync_copy(zbuf, acc1_sh.at[sl])
        pltpu.sync_copy(zbuf, acc2_sh.at[sl])

        pltpu.sync_copy(dst_h.at[wid], dst_v)
        pltpu.sync_copy(ew_h.at[wid], ew_v)
        pltpu.sync_copy(one_h.at[wid], one_v)
        plsc.subcore_barrier()

        def blk(j, _):
            idx = dst_v.at[j]
            pltpu.sync_copy(ew_v.at[j], acc1_sh.at[idx], add=True)
            pltpu.sync_copy(one_v.at[j], acc2_sh.at[idx], add=True)
            return 0

        lax.fori_loop(0, NBLK, blk, 0)
        plsc.subcore_barrier()

        pltpu.sync_copy(acc1_sh.at[sl], out_h.at[lc, 0, sl])
        pltpu.sync_copy(acc2_sh.at[sl], out_h.at[lc, 1, sl])

    return deg_k(dst_p, ew_p, one_p)


NB = 2


def _sc_aggregate(src_p, dst_p, ew_p, feat, width):
    scratch = [
        pltpu.VMEM((NBLK, B), jnp.int32),
        pltpu.VMEM((NBLK, B), jnp.float32),
        pltpu.VMEM((NBLK, B), jnp.int32),
        pltpu.VMEM((NB, B, width), jnp.float32),
        pltpu.VMEM_SHARED((NPAD, width), jnp.float32),
        pltpu.VMEM_SHARED((N, width), jnp.float32),
    ] + [pltpu.SemaphoreType.DMA] * (2 * NB)
    scale = ew_p is not None
    params = (
        None
        if width % 128 == 0
        else pltpu.CompilerParams(use_tc_tiling_on_sc=False)
    )

    @functools.partial(
        pl.kernel,
        out_type=jax.ShapeDtypeStruct((NC, NPAD, width), jnp.float32),
        mesh=_sc_mesh(),
        scratch_types=scratch,
        compiler_params=params,
    )
    def agg_k(src_h, ew_h, dst_h, feat_h, out_h, src_v, ew_v, dst_v, rows, acc_sh, feat_sh, *sems):
        gsem = sems[:NB]
        ssem = sems[NB:]
        lc = lax.axis_index("c")
        ls = lax.axis_index("s")
        wid = ls * NC + lc

        zv = jnp.zeros((L,), jnp.float32)

        def zb(i, _):
            for k in range(width // L):
                rows[0, i, pl.ds(k * L, L)] = zv
            return 0

        lax.fori_loop(0, B, zb, 0)

        def zs(t, _):
            pltpu.sync_copy(rows.at[0], acc_sh.at[pl.ds(ls * SLAB + t * B, B)])
            return 0

        lax.fori_loop(0, SLAB // B, zs, 0)

        pltpu.sync_copy(src_h.at[wid], src_v)
        pltpu.sync_copy(dst_h.at[wid], dst_v)
        if scale:
            pltpu.sync_copy(ew_h.at[wid], ew_v)
        fsl = pl.ds(ls * (N // NS), N // NS)
        pltpu.sync_copy(feat_h.at[fsl], feat_sh.at[fsl])
        plsc.subcore_barrier()

        def g_issue(j, b):
            pltpu.async_copy(feat_sh.at[src_v.at[j]], rows.at[b], gsem[b])

        def g_wait(j, b):
            pltpu.make_async_copy(feat_sh.at[src_v.at[j]], rows.at[b], gsem[b]).wait()

        def s_issue(j, b):
            pltpu.async_copy(rows.at[b], acc_sh.at[dst_v.at[j]], ssem[b], add=True)

        def s_wait(j, b):
            pltpu.make_async_copy(rows.at[b], acc_sh.at[dst_v.at[j]], ssem[b]).wait()

        def do_scale(j, b):
            def scale_g(g, _):
                base = g * L
                wvec = ew_v[j, pl.ds(base, L)]
                for e in range(L):
                    w = wvec[e]
                    for k in range(width // L):
                        sl = pl.ds(k * L, L)
                        rows[b, base + e, sl] = rows[b, base + e, sl] * w
                return 0

            lax.fori_loop(0, B // L, scale_g, 0)

        g_issue(0, 0)

        def turn(t, _):
            for b in range(NB):
                j = t * NB + b
                bn = (b + 1) % NB

                @pl.when(j >= NB - 1)
                def _():
                    s_wait(j - (NB - 1), bn)

                @pl.when(j <= NBLK - 2)
                def _():
                    g_issue(j + 1, bn)

                g_wait(j, b)
                if scale:
                    do_scale(j, b)
                s_issue(j, b)
            return 0

        lax.fori_loop(0, NBLK // NB, turn, 0)
        for j in range(NBLK - NB + 1, NBLK):
            s_wait(j, j % NB)
        plsc.subcore_barrier()

        def co(t, _):
            sl = pl.ds(ls * SLAB + t * B, B)
            pltpu.sync_copy(acc_sh.at[sl], out_h.at[lc, sl])
            return 0

        lax.fori_loop(0, SLAB // B, co, 0)

    if ew_p is None:
        ew_p = src_p
    return agg_k(src_p, ew_p, dst_p, feat)


def _tc1(x, W1, degp):
    def body(x_ref, w_ref, degp_ref, hpa_ref, hpb_ref, d1_ref, d2_ref):
        dgp = degp_ref[...]
        deg1 = dgp[0] + dgp[2] + 1.0
        deg2 = dgp[1] + dgp[3] + 1.0
        d1 = lax.rsqrt(deg1)
        d2 = lax.rsqrt(deg2)
        h = jnp.dot(x_ref[...], w_ref[...], preferred_element_type=jnp.float32)
        hp = h * d1
        hpa_ref[...] = hp[:, : D // 2]
        hpb_ref[...] = hp[:, D // 2 :]
        d1_ref[...] = d1
        d2_ref[...] = d2

    return pl.pallas_call(
        body,
        grid=(N // RBLK,),
        in_specs=[
            pl.BlockSpec((RBLK, D), lambda i: (i, 0)),
            pl.BlockSpec((D, D), lambda i: (0, 0)),
            pl.BlockSpec((NC * 2, RBLK, 1), lambda i: (0, i, 0)),
        ],
        out_specs=[
            pl.BlockSpec((RBLK, D // 2), lambda i: (i, 0)),
            pl.BlockSpec((RBLK, D // 2), lambda i: (i, 0)),
            pl.BlockSpec((RBLK, 1), lambda i: (i, 0)),
            pl.BlockSpec((RBLK, 1), lambda i: (i, 0)),
        ],
        out_shape=[
            jax.ShapeDtypeStruct((N, D // 2), jnp.float32),
            jax.ShapeDtypeStruct((N, D // 2), jnp.float32),
            jax.ShapeDtypeStruct((N, 1), jnp.float32),
            jax.ShapeDtypeStruct((N, 1), jnp.float32),
        ],
    )(x, W1, degp)


def _tc2(acc1a, acc1b, hpa, hpb, d1, d2, b1, W2p):
    def body(aa_ref, ab_ref, hpa_ref, hpb_ref, d1_ref, d2_ref, b1_ref, w2_ref, h2p_ref):
        aa = aa_ref[...]
        ab = ab_ref[...]
        acc = jnp.concatenate([aa[0] + aa[1], ab[0] + ab[1]], axis=1)
        hp = jnp.concatenate([hpa_ref[...], hpb_ref[...]], axis=1)
        out1 = jnp.maximum(d1_ref[...] * (acc + hp) + b1_ref[...], 0.0)
        h2 = jnp.dot(out1, w2_ref[...], preferred_element_type=jnp.float32)
        h2p_ref[...] = h2 * d2_ref[...]

    half = pl.BlockSpec((NC, RBLK, D // 2), lambda i: (0, i, 0))
    feat = pl.BlockSpec((RBLK, D // 2), lambda i: (i, 0))
    return pl.pallas_call(
        body,
        grid=(N // RBLK,),
        in_specs=[
            half,
            half,
            feat,
            feat,
            pl.BlockSpec((RBLK, 1), lambda i: (i, 0)),
            pl.BlockSpec((RBLK, 1), lambda i: (i, 0)),
            pl.BlockSpec((1, D), lambda i: (0, 0)),
            pl.BlockSpec((D, D2), lambda i: (0, 0)),
        ],
        out_specs=pl.BlockSpec((RBLK, D2), lambda i: (i, 0)),
        out_shape=jax.ShapeDtypeStruct((N, D2), jnp.float32),
    )(acc1a, acc1b, hpa, hpb, d1, d2, b1, W2p)


def _tc3(acc2, h2p, d2, b2p):
    def body(acc_ref, h2p_ref, d2_ref, b2_ref, out_ref):
        a = acc_ref[...]
        full = d2_ref[...] * (a[0] + a[1] + h2p_ref[...]) + b2_ref[...]
        out_ref[...] = full[:, :C]

    return pl.pallas_call(
        body,
        grid=(N // RBLK,),
        in_specs=[
            pl.BlockSpec((NC, RBLK, D2), lambda i: (0, i, 0)),
            pl.BlockSpec((RBLK, D2), lambda i: (i, 0)),
            pl.BlockSpec((RBLK, 1), lambda i: (i, 0)),
            pl.BlockSpec((1, D2), lambda i: (0, 0)),
        ],
        out_specs=pl.BlockSpec((RBLK, C), lambda i: (i, 0)),
        out_shape=jax.ShapeDtypeStruct((N, C), jnp.float32),
    )(acc2, h2p, d2, b2p)


def kernel(x, edge_index, edge_weight, W1, b1, W2, b2):
    src = edge_index[0].astype(jnp.int32)
    dst = edge_index[1].astype(jnp.int32)
    ew = edge_weight.astype(jnp.float32)

    ppw = EPW - E // NW
    pad_dst = jnp.broadcast_to(N + jnp.arange(ppw, dtype=jnp.int32), (NW, ppw))

    def _pack_i(a, pad_val):
        return jnp.concatenate(
            [a.reshape(NW, E // NW), pad_val], axis=1
        ).reshape(NW, NBLK, B)

    src_p = _pack_i(src, jnp.zeros((NW, ppw), jnp.int32))
    dst_p = _pack_i(dst, pad_dst)
    ew_p = _pack_i(ew, jnp.zeros((NW, ppw), jnp.float32))
    one_p = _pack_i(jnp.ones((E,), jnp.float32), jnp.zeros((NW, ppw), jnp.float32))

    W2p = jnp.pad(W2.astype(jnp.float32), ((0, 0), (0, D2 - C)))
    b1r = b1.astype(jnp.float32).reshape(1, D)
    b2p = jnp.pad(b2.astype(jnp.float32), (0, D2 - C)).reshape(1, D2)

    degp = _sc_degrees(dst_p, ew_p, one_p)
    degp4 = degp.reshape(NC * 2, NPAD)[:, :, None]
    hpa, hpb, d1, d2 = _tc1(x.astype(jnp.float32), W1.astype(jnp.float32), degp4)
    acc1a = _sc_aggregate(src_p, dst_p, ew_p, hpa, D // 2)
    acc1b = _sc_aggregate(src_p, dst_p, ew_p, hpb, D // 2)
    h2p = _tc2(acc1a, acc1b, hpa, hpb, d1, d2, b1r, W2p)
    acc2 = _sc_aggregate(src_p, dst_p, None, h2p, D2)
    return _tc3(acc2, h2p, d2, b2p)


# speedup vs baseline: 2.0334x; 1.5314x over previous
import functools

import jax
import jax.numpy as jnp
from jax import lax
from jax.experimental import pallas as pl
from jax.experimental.pallas import tpu as pltpu
from jax.experimental.pallas import tpu_sc as plsc

N = 10000
E = 320000
D = 128
C = 40
D2 = 48
NC, NS, L = 2, 16, 16
NW = NC * NS
B = 128
EPW = 10240
NBLK = EPW // B
NPAD = 10240
SLAB = NPAD // NS
RBLK = 2000


def _zero_rows(ref, nrows, width):
    zv = jnp.zeros((L,), jnp.float32)

    def body(i, _):
        for k in range(width // L):
            ref[i, pl.ds(k * L, L)] = zv
        return 0

    lax.fori_loop(0, nrows, body, 0)


def _sc_mesh():
    return plsc.VectorSubcoreMesh(core_axis_name="c", subcore_axis_name="s")


def _sc_degrees(dst_p, ew_p, one_p):
    @functools.partial(
        pl.kernel,
        out_type=jax.ShapeDtypeStruct((NC, 2, NPAD), jnp.float32),
        mesh=_sc_mesh(),
        scratch_types=[
            pltpu.VMEM((NBLK, B), jnp.int32),
            pltpu.VMEM((NBLK, B), jnp.float32),
            pltpu.VMEM((NBLK, B), jnp.float32),
            pltpu.VMEM((SLAB,), jnp.float32),
            pltpu.VMEM_SHARED((NPAD,), jnp.float32),
            pltpu.VMEM_SHARED((NPAD,), jnp.float32),
        ],
    )
    def deg_k(dst_h, ew_h, one_h, out_h, dst_v, ew_v, one_v, zbuf, acc1_sh, acc2_sh):
        lc = lax.axis_index("c")
        ls = lax.axis_index("s")
        wid = ls * NC + lc

        zv = jnp.zeros((L,), jnp.float32)

        def zb(i, _):
            zbuf[pl.ds(i * L, L)] = zv
            return 0

        lax.fori_loop(0, SLAB // L, zb, 0)
        sl = pl.ds(ls * SLAB, SLAB)
        pltpu.sync_copy(zbuf, acc1_sh.at[sl])
        pltpu.sync_copy(zbuf, acc2_sh.at[sl])

        pltpu.sync_copy(dst_h.at[wid], dst_v)
        pltpu.sync_copy(ew_h.at[wid], ew_v)
        pltpu.sync_copy(one_h.at[wid], one_v)
        plsc.subcore_barrier()

        def blk(j, _):
            idx = dst_v.at[j]
            pltpu.sync_copy(ew_v.at[j], acc1_sh.at[idx], add=True)
            pltpu.sync_copy(one_v.at[j], acc2_sh.at[idx], add=True)
            return 0

        lax.fori_loop(0, NBLK, blk, 0)
        plsc.subcore_barrier()

        pltpu.sync_copy(acc1_sh.at[sl], out_h.at[lc, 0, sl])
        pltpu.sync_copy(acc2_sh.at[sl], out_h.at[lc, 1, sl])

    return deg_k(dst_p, ew_p, one_p)


NB = 2


def _sc_aggregate(src_p, dst_p, ew_p, feats, width):
    npass = len(feats)
    scratch = [
        pltpu.VMEM((NBLK, B), jnp.int32),
        pltpu.VMEM((NBLK, B), jnp.float32),
        pltpu.VMEM((NBLK, B), jnp.int32),
        pltpu.VMEM((NB, B, width), jnp.float32),
        pltpu.VMEM_SHARED((NPAD, width), jnp.float32),
        pltpu.VMEM_SHARED((N, width), jnp.float32),
    ] + [pltpu.SemaphoreType.DMA] * (2 * NB)
    scale = ew_p is not None
    params = (
        None
        if width % 128 == 0
        else pltpu.CompilerParams(use_tc_tiling_on_sc=False)
    )

    @functools.partial(
        pl.kernel,
        out_type=[jax.ShapeDtypeStruct((NC, NPAD, width), jnp.float32)] * npass,
        mesh=_sc_mesh(),
        scratch_types=scratch,
        compiler_params=params,
    )
    def agg_k(src_h, ew_h, dst_h, *rest):
        feat_hs = rest[:npass]
        out_hs = rest[npass : 2 * npass]
        src_v, ew_v, dst_v, rows, acc_sh, feat_sh = rest[2 * npass : 2 * npass + 6]
        sems = rest[2 * npass + 6 :]
        gsem = sems[:NB]
        ssem = sems[NB:]
        lc = lax.axis_index("c")
        ls = lax.axis_index("s")
        wid = ls * NC + lc
        fsl = pl.ds(ls * (N // NS), N // NS)

        pltpu.sync_copy(src_h.at[wid], src_v)
        pltpu.sync_copy(dst_h.at[wid], dst_v)
        if scale:
            pltpu.sync_copy(ew_h.at[wid], ew_v)

        zv = jnp.zeros((L,), jnp.float32)

        def g_issue(j, b):
            pltpu.async_copy(feat_sh.at[src_v.at[j]], rows.at[b], gsem[b])

        def g_wait(j, b):
            pltpu.make_async_copy(feat_sh.at[src_v.at[j]], rows.at[b], gsem[b]).wait()

        def s_issue(j, b):
            pltpu.async_copy(rows.at[b], acc_sh.at[dst_v.at[j]], ssem[b], add=True)

        def s_wait(j, b):
            pltpu.make_async_copy(rows.at[b], acc_sh.at[dst_v.at[j]], ssem[b]).wait()

        def do_scale(j, b):
            def scale_g(g, _):
                base = g * L
                wvec = ew_v[j, pl.ds(base, L)]
                for e0 in range(0, L, 4):
                    ws = [wvec[e0 + i] for i in range(4)]
                    for k in range(width // L):
                        sl = pl.ds(k * L, L)
                        vals = [rows[b, base + e0 + i, sl] for i in range(4)]
                        for i in range(4):
                            rows[b, base + e0 + i, sl] = vals[i] * ws[i]
                return 0

            lax.fori_loop(0, B // L, scale_g, 0)

        for p in range(npass):
            def zb(i, _):
                for k in range(width // L):
                    rows[0, i, pl.ds(k * L, L)] = zv
                return 0

            lax.fori_loop(0, B, zb, 0)

            def zs(t, _):
                pltpu.sync_copy(rows.at[0], acc_sh.at[pl.ds(ls * SLAB + t * B, B)])
                return 0

            lax.fori_loop(0, SLAB // B, zs, 0)
            pltpu.sync_copy(feat_hs[p].at[fsl], feat_sh.at[fsl])
            plsc.subcore_barrier()

            g_issue(0, 0)

            def turn(t, _):
                for b in range(NB):
                    j = t * NB + b
                    bn = (b + 1) % NB

                    @pl.when(j >= NB - 1)
                    def _():
                        s_wait(j - (NB - 1), bn)

                    @pl.when(j <= NBLK - 2)
                    def _():
                        g_issue(j + 1, bn)

                    g_wait(j, b)
                    if scale:
                        do_scale(j, b)
                    s_issue(j, b)
                return 0

            lax.fori_loop(0, NBLK // NB, turn, 0)
            for j in range(NBLK - NB + 1, NBLK):
                s_wait(j, j % NB)
            plsc.subcore_barrier()

            def co(t, _):
                sl = pl.ds(ls * SLAB + t * B, B)
                pltpu.sync_copy(acc_sh.at[sl], out_hs[p].at[lc, sl])
                return 0

            lax.fori_loop(0, SLAB // B, co, 0)
            plsc.subcore_barrier()

    if ew_p is None:
        ew_p = src_p
    return agg_k(src_p, ew_p, dst_p, *feats)


def _tc1(x, W1, degp):
    def body(x_ref, w_ref, degp_ref, hpa_ref, hpb_ref, d1_ref, d2_ref):
        dgp = degp_ref[...]
        deg1 = dgp[0] + dgp[2] + 1.0
        deg2 = dgp[1] + dgp[3] + 1.0
        d1 = lax.rsqrt(deg1)
        d2 = lax.rsqrt(deg2)
        h = jnp.dot(x_ref[...], w_ref[...], preferred_element_type=jnp.float32)
        hp = h * d1
        hpa_ref[...] = hp[:, : D // 2]
        hpb_ref[...] = hp[:, D // 2 :]
        d1_ref[...] = d1
        d2_ref[...] = d2

    return pl.pallas_call(
        body,
        grid=(N // RBLK,),
        in_specs=[
            pl.BlockSpec((RBLK, D), lambda i: (i, 0)),
            pl.BlockSpec((D, D), lambda i: (0, 0)),
            pl.BlockSpec((NC * 2, RBLK, 1), lambda i: (0, i, 0)),
        ],
        out_specs=[
            pl.BlockSpec((RBLK, D // 2), lambda i: (i, 0)),
            pl.BlockSpec((RBLK, D // 2), lambda i: (i, 0)),
            pl.BlockSpec((RBLK, 1), lambda i: (i, 0)),
            pl.BlockSpec((RBLK, 1), lambda i: (i, 0)),
        ],
        out_shape=[
            jax.ShapeDtypeStruct((N, D // 2), jnp.float32),
            jax.ShapeDtypeStruct((N, D // 2), jnp.float32),
            jax.ShapeDtypeStruct((N, 1), jnp.float32),
            jax.ShapeDtypeStruct((N, 1), jnp.float32),
        ],
    )(x, W1, degp)


def _tc2(acc1a, acc1b, hpa, hpb, d1, d2, b1, W2p):
    def body(aa_ref, ab_ref, hpa_ref, hpb_ref, d1_ref, d2_ref, b1_ref, w2_ref, h2p_ref):
        aa = aa_ref[...]
        ab = ab_ref[...]
        acc = jnp.concatenate([aa[0] + aa[1], ab[0] + ab[1]], axis=1)
        hp = jnp.concatenate([hpa_ref[...], hpb_ref[...]], axis=1)
        out1 = jnp.maximum(d1_ref[...] * (acc + hp) + b1_ref[...], 0.0)
        h2 = jnp.dot(out1, w2_ref[...], preferred_element_type=jnp.float32)
        h2p_ref[...] = h2 * d2_ref[...]

    half = pl.BlockSpec((NC, RBLK, D // 2), lambda i: (0, i, 0))
    feat = pl.BlockSpec((RBLK, D // 2), lambda i: (i, 0))
    return pl.pallas_call(
        body,
        grid=(N // RBLK,),
        in_specs=[
            half,
            half,
            feat,
            feat,
            pl.BlockSpec((RBLK, 1), lambda i: (i, 0)),
            pl.BlockSpec((RBLK, 1), lambda i: (i, 0)),
            pl.BlockSpec((1, D), lambda i: (0, 0)),
            pl.BlockSpec((D, D2), lambda i: (0, 0)),
        ],
        out_specs=pl.BlockSpec((RBLK, D2), lambda i: (i, 0)),
        out_shape=jax.ShapeDtypeStruct((N, D2), jnp.float32),
    )(acc1a, acc1b, hpa, hpb, d1, d2, b1, W2p)


def _tc3(acc2, h2p, d2, b2p):
    def body(acc_ref, h2p_ref, d2_ref, b2_ref, out_ref):
        a = acc_ref[...]
        full = d2_ref[...] * (a[0] + a[1] + h2p_ref[...]) + b2_ref[...]
        out_ref[...] = full[:, :C]

    return pl.pallas_call(
        body,
        grid=(N // RBLK,),
        in_specs=[
            pl.BlockSpec((NC, RBLK, D2), lambda i: (0, i, 0)),
            pl.BlockSpec((RBLK, D2), lambda i: (i, 0)),
            pl.BlockSpec((RBLK, 1), lambda i: (i, 0)),
            pl.BlockSpec((1, D2), lambda i: (0, 0)),
        ],
        out_specs=pl.BlockSpec((RBLK, C), lambda i: (i, 0)),
        out_shape=jax.ShapeDtypeStruct((N, C), jnp.float32),
    )(acc2, h2p, d2, b2p)


def kernel(x, edge_index, edge_weight, W1, b1, W2, b2):
    src = edge_index[0].astype(jnp.int32)
    dst = edge_index[1].astype(jnp.int32)
    ew = edge_weight.astype(jnp.float32)

    ppw = EPW - E // NW
    pad_dst = jnp.broadcast_to(N + jnp.arange(ppw, dtype=jnp.int32), (NW, ppw))

    def _pack_i(a, pad_val):
        return jnp.concatenate(
            [a.reshape(NW, E // NW), pad_val], axis=1
        ).reshape(NW, NBLK, B)

    src_p = _pack_i(src, jnp.zeros((NW, ppw), jnp.int32))
    dst_p = _pack_i(dst, pad_dst)
    ew_p = _pack_i(ew, jnp.zeros((NW, ppw), jnp.float32))
    one_p = _pack_i(jnp.ones((E,), jnp.float32), jnp.zeros((NW, ppw), jnp.float32))

    W2p = jnp.pad(W2.astype(jnp.float32), ((0, 0), (0, D2 - C)))
    b1r = b1.astype(jnp.float32).reshape(1, D)
    b2p = jnp.pad(b2.astype(jnp.float32), (0, D2 - C)).reshape(1, D2)

    degp = _sc_degrees(dst_p, ew_p, one_p)
    degp4 = degp.reshape(NC * 2, NPAD)[:, :, None]
    hpa, hpb, d1, d2 = _tc1(x.astype(jnp.float32), W1.astype(jnp.float32), degp4)
    acc1a, acc1b = _sc_aggregate(src_p, dst_p, ew_p, [hpa, hpb], D // 2)
    h2p = _tc2(acc1a, acc1b, hpa, hpb, d1, d2, b1r, W2p)
    (acc2,) = _sc_aggregate(src_p, dst_p, None, [h2p], D2)
    return _tc3(acc2, h2p, d2, b2p)
